# baseline (speedup 1.0000x reference)
import functools

import jax
import jax.numpy as jnp
from jax import lax
from jax.experimental import pallas as pl
from jax.experimental.pallas import tpu as pltpu
from jax.experimental.pallas import tpu_sc as plsc

N_NODES = 10000
N_EDGES = 320000
D = 128

NC = 2
NS = 16
NW = NC * NS
K = 128

CT = -(-N_EDGES // K)
CPW = 80
CT_PAD = CPW * NW
E_PAD = CT_PAD * K

NP = 10112
RPT = NP // NS

_mesh = plsc.VectorSubcoreMesh(core_axis_name="c", subcore_axis_name="s")


@functools.partial(
    pl.kernel,
    out_type=jax.ShapeDtypeStruct((NC, NP, D), jnp.float32),
    mesh=_mesh,
    scratch_types=[
        pltpu.VMEM((CPW, K), jnp.int32),
        pltpu.VMEM((K, D), jnp.float32),
        pltpu.VMEM_SHARED((NP, D), jnp.float32),
    ],
)
def _sc_degree(dst2d, onesd, zerosd, out, dst_v, ones_v, acc):
    c = lax.axis_index("c")
    s = lax.axis_index("s")
    w = c * NS + s
    pltpu.sync_copy(dst2d.at[pl.ds(w * CPW, CPW)], dst_v)
    pltpu.sync_copy(onesd, ones_v)
    pltpu.sync_copy(zerosd, acc.at[pl.ds(s * RPT, RPT)])
    plsc.subcore_barrier()

    def body(j, carry):
        pltpu.sync_copy(ones_v, acc.at[dst_v.at[j]], add=True)
        return carry

    lax.fori_loop(0, CPW, body, 0)
    plsc.subcore_barrier()
    pltpu.sync_copy(acc.at[pl.ds(s * RPT, RPT)], out.at[c, pl.ds(s * RPT, RPT)])


@functools.partial(
    pl.kernel,
    out_type=jax.ShapeDtypeStruct((NC, NP, D), jnp.float32),
    mesh=_mesh,
    scratch_types=[
        pltpu.VMEM((CPW, K), jnp.int32),
        pltpu.VMEM((CPW, K), jnp.int32),
        pltpu.VMEM((K, D), jnp.float32),
        pltpu.VMEM_SHARED((NP, D), jnp.float32),
    ],
)
def _sc_propagate(u, src2d, dst2d, zerosd, out, src_v, dst_v, rows_v, acc):
    c = lax.axis_index("c")
    s = lax.axis_index("s")
    w = c * NS + s
    pltpu.sync_copy(src2d.at[pl.ds(w * CPW, CPW)], src_v)
    pltpu.sync_copy(dst2d.at[pl.ds(w * CPW, CPW)], dst_v)
    pltpu.sync_copy(zerosd, acc.at[pl.ds(s * RPT, RPT)])
    plsc.subcore_barrier()

    def body(j, carry):
        pltpu.sync_copy(u.at[src_v.at[j]], rows_v)
        pltpu.sync_copy(rows_v, acc.at[dst_v.at[j]], add=True)
        return carry

    lax.fori_loop(0, CPW, body, 0)
    plsc.subcore_barrier()
    pltpu.sync_copy(acc.at[pl.ds(s * RPT, RPT)], out.at[c, pl.ds(s * RPT, RPT)])


def _tc_first(x_pad, W0, d0, d1):
    def body(x_ref, w_ref, d0_ref, d1_ref, u_ref, dis_ref):
        deg = d0_ref[:, :1] + d1_ref[:, :1]
        dis = jnp.where(deg > 0.0, lax.rsqrt(jnp.maximum(deg, 1.0)), 0.0)
        u_ref[...] = dis * jnp.dot(x_ref[...], w_ref[...],
                                   preferred_element_type=jnp.float32)
        dis_ref[...] = dis

    return pl.pallas_call(
        body,
        out_shape=[
            jax.ShapeDtypeStruct((NP, D), jnp.float32),
            jax.ShapeDtypeStruct((NP, 1), jnp.float32),
        ],
    )(x_pad, W0, d0, d1)


def _tc_mid(s0, s1, dis, b, W):
    def body(s0_ref, s1_ref, dis_ref, b_ref, w_ref, u_ref):
        dis = dis_ref[...]
        h = jnp.maximum(dis * (s0_ref[...] + s1_ref[...]) + b_ref[...], 0.0)
        u_ref[...] = dis * jnp.dot(h, w_ref[...],
                                   preferred_element_type=jnp.float32)

    return pl.pallas_call(
        body,
        out_shape=jax.ShapeDtypeStruct((NP, D), jnp.float32),
    )(s0, s1, dis, b, W)


def _tc_final(s0, s1, dis, b):
    def body(s0_ref, s1_ref, dis_ref, b_ref, o_ref):
        o_ref[...] = dis_ref[...] * (s0_ref[...] + s1_ref[...]) + b_ref[...]

    return pl.pallas_call(
        body,
        out_shape=jax.ShapeDtypeStruct((NP, D), jnp.float32),
    )(s0, s1, dis, b)


def kernel(x, edge_index, W0, b0, W1, b1, W2, b2, W3, b3):
    src = edge_index[0]
    dst = edge_index[1]
    pad = E_PAD - N_EDGES
    src_p = jnp.concatenate([src, jnp.full((pad,), N_NODES, jnp.int32)])
    dst_p = jnp.concatenate(
        [dst, N_NODES + (jnp.arange(pad, dtype=jnp.int32) % (NP - N_NODES))])
    order = jnp.argsort(src_p)
    src2d = src_p[order].reshape(CT_PAD, K)
    dst2d = dst_p[order].reshape(CT_PAD, K)
    perm = jnp.arange(CT_PAD).reshape(CPW, NW).T.reshape(-1)
    src2d = src2d[perm]
    dst2d = dst2d[perm]

    onesd = jnp.tile(
        (jnp.arange(D) == 0).astype(jnp.float32)[None, :], (K, 1))
    zerosd = jnp.zeros((RPT, D), jnp.float32)
    x_pad = jnp.pad(x, ((0, NP - N_NODES), (0, 0)))

    degp = _sc_degree(dst2d, onesd, zerosd)
    u, dis = _tc_first(x_pad, W0, degp[0, :, :1], degp[1, :, :1])

    bs = [b0.reshape(1, D), b1.reshape(1, D), b2.reshape(1, D), b3.reshape(1, D)]
    Ws = [W1, W2, W3]
    for i in range(3):
        sp = _sc_propagate(u, src2d, dst2d, zerosd)
        u = _tc_mid(sp[0], sp[1], dis, bs[i], Ws[i])
    sp = _sc_propagate(u, src2d, dst2d, zerosd)
    out = _tc_final(sp[0], sp[1], dis, bs[3])
    return out[:N_NODES]


# speedup vs baseline: 3.2398x; 3.2398x over previous
import functools

import jax
import jax.numpy as jnp
from jax import lax
from jax.experimental import pallas as pl
from jax.experimental.pallas import tpu as pltpu
from jax.experimental.pallas import tpu_sc as plsc

N_NODES = 10000
N_EDGES = 320000
D = 128

NC = 2
NS = 16
NW = NC * NS
K = 128

CT = -(-N_EDGES // K)
CPW = 80
CT_PAD = CPW * NW
E_PAD = CT_PAD * K

NP = 10112
RPT = NP // NS

_mesh = plsc.VectorSubcoreMesh(core_axis_name="c", subcore_axis_name="s")


@functools.partial(
    pl.kernel,
    out_type=jax.ShapeDtypeStruct((NC, NP, D), jnp.float32),
    mesh=_mesh,
    scratch_types=[
        pltpu.VMEM((CPW, K), jnp.int32),
        pltpu.VMEM((K, D), jnp.float32),
        pltpu.VMEM_SHARED((NP, D), jnp.float32),
    ],
)
def _sc_degree(dst2d, onesd, zerosd, out, dst_v, ones_v, acc):
    c = lax.axis_index("c")
    s = lax.axis_index("s")
    w = c * NS + s
    pltpu.sync_copy(dst2d.at[pl.ds(w * CPW, CPW)], dst_v)
    pltpu.sync_copy(onesd, ones_v)
    pltpu.sync_copy(zerosd, acc.at[pl.ds(s * RPT, RPT)])
    plsc.subcore_barrier()

    def body(j, carry):
        pltpu.sync_copy(ones_v, acc.at[dst_v.at[j]], add=True)
        return carry

    lax.fori_loop(0, CPW, body, 0)
    plsc.subcore_barrier()
    pltpu.sync_copy(acc.at[pl.ds(s * RPT, RPT)], out.at[c, pl.ds(s * RPT, RPT)])


@functools.partial(
    pl.kernel,
    out_type=jax.ShapeDtypeStruct((NC, NP, D), jnp.float32),
    mesh=_mesh,
    scratch_types=[
        pltpu.VMEM((CPW, K), jnp.int32),
        pltpu.VMEM((CPW, K), jnp.int32),
        pltpu.VMEM((K, D), jnp.float32),
        pltpu.VMEM_SHARED((NP, D), jnp.float32),
    ],
)
def _sc_propagate(u, src2d, dst2d, zerosd, out, src_v, dst_v, rows_v, acc):
    c = lax.axis_index("c")
    s = lax.axis_index("s")
    w = c * NS + s
    pltpu.sync_copy(src2d.at[pl.ds(w * CPW, CPW)], src_v)
    pltpu.sync_copy(dst2d.at[pl.ds(w * CPW, CPW)], dst_v)
    pltpu.sync_copy(zerosd, acc.at[pl.ds(s * RPT, RPT)])
    plsc.subcore_barrier()

    def body(j, carry):
        pltpu.sync_copy(u.at[src_v.at[j]], rows_v)
        pltpu.sync_copy(rows_v, acc.at[dst_v.at[j]], add=True)
        return carry

    lax.fori_loop(0, CPW, body, 0)
    plsc.subcore_barrier()
    pltpu.sync_copy(acc.at[pl.ds(s * RPT, RPT)], out.at[c, pl.ds(s * RPT, RPT)])


def _tc_first(x_pad, W0, d0, d1):
    def body(x_ref, w_ref, d0_ref, d1_ref, u_ref, dis_ref):
        deg = d0_ref[:, :1] + d1_ref[:, :1]
        dis = jnp.where(deg > 0.0, lax.rsqrt(jnp.maximum(deg, 1.0)), 0.0)
        u_ref[...] = dis * jnp.dot(x_ref[...], w_ref[...],
                                   preferred_element_type=jnp.float32)
        dis_ref[...] = dis

    return pl.pallas_call(
        body,
        out_shape=[
            jax.ShapeDtypeStruct((NP, D), jnp.float32),
            jax.ShapeDtypeStruct((NP, 1), jnp.float32),
        ],
    )(x_pad, W0, d0, d1)


def _tc_mid(s0, s1, dis, b, W):
    def body(s0_ref, s1_ref, dis_ref, b_ref, w_ref, u_ref):
        dis = dis_ref[...]
        h = jnp.maximum(dis * (s0_ref[...] + s1_ref[...]) + b_ref[...], 0.0)
        u_ref[...] = dis * jnp.dot(h, w_ref[...],
                                   preferred_element_type=jnp.float32)

    return pl.pallas_call(
        body,
        out_shape=jax.ShapeDtypeStruct((NP, D), jnp.float32),
    )(s0, s1, dis, b, W)


def _tc_final(s0, s1, dis, b):
    def body(s0_ref, s1_ref, dis_ref, b_ref, o_ref):
        o_ref[...] = dis_ref[...] * (s0_ref[...] + s1_ref[...]) + b_ref[...]

    return pl.pallas_call(
        body,
        out_shape=jax.ShapeDtypeStruct((NP, D), jnp.float32),
    )(s0, s1, dis, b)


def kernel(x, edge_index, W0, b0, W1, b1, W2, b2, W3, b3):
    src = edge_index[0]
    dst = edge_index[1]
    pad = E_PAD - N_EDGES
    pad_iota = jnp.arange(pad, dtype=jnp.int32) % (NP - N_NODES)
    src2d = jnp.concatenate([src, N_NODES + pad_iota]).reshape(CT_PAD, K)
    dst2d = jnp.concatenate([dst, N_NODES + pad_iota]).reshape(CT_PAD, K)
    perm = jnp.arange(CT_PAD).reshape(CPW, NW).T.reshape(-1)
    src2d = src2d[perm]
    dst2d = dst2d[perm]

    onesd = jnp.tile(
        (jnp.arange(D) == 0).astype(jnp.float32)[None, :], (K, 1))
    zerosd = jnp.zeros((RPT, D), jnp.float32)
    x_pad = jnp.pad(x, ((0, NP - N_NODES), (0, 0)))

    degp = _sc_degree(dst2d, onesd, zerosd)
    u, dis = _tc_first(x_pad, W0, degp[0, :, :1], degp[1, :, :1])

    bs = [b0.reshape(1, D), b1.reshape(1, D), b2.reshape(1, D), b3.reshape(1, D)]
    Ws = [W1, W2, W3]
    for i in range(3):
        sp = _sc_propagate(u, src2d, dst2d, zerosd)
        u = _tc_mid(sp[0], sp[1], dis, bs[i], Ws[i])
    sp = _sc_propagate(u, src2d, dst2d, zerosd)
    out = _tc_final(sp[0], sp[1], dis, bs[3])
    return out[:N_NODES]


# speedup vs baseline: 4.4965x; 1.3879x over previous
import functools

import jax
import jax.numpy as jnp
from jax import lax
from jax.experimental import pallas as pl
from jax.experimental.pallas import tpu as pltpu
from jax.experimental.pallas import tpu_sc as plsc

N_NODES = 10000
N_EDGES = 320000
D = 128

NC = 2
NS = 16
NW = NC * NS
K = 128

CT = -(-N_EDGES // K)
CPW = 80
CT_PAD = CPW * NW
E_PAD = CT_PAD * K

NP = 10112
RPT = NP // NS

_mesh = plsc.VectorSubcoreMesh(core_axis_name="c", subcore_axis_name="s")


@functools.partial(
    pl.kernel,
    out_type=jax.ShapeDtypeStruct((NC, NP, D), jnp.float32),
    mesh=_mesh,
    scratch_types=[
        pltpu.VMEM((CPW, K), jnp.int32),
        pltpu.VMEM((K, D), jnp.float32),
        pltpu.VMEM_SHARED((NP, D), jnp.float32),
    ],
)
def _sc_degree(dst2d, onesd, zerosd, out, dst_v, ones_v, acc):
    c = lax.axis_index("c")
    s = lax.axis_index("s")
    w = c * NS + s
    pltpu.sync_copy(dst2d.at[pl.ds(w * CPW, CPW)], dst_v)
    pltpu.sync_copy(onesd, ones_v)
    pltpu.sync_copy(zerosd, acc.at[pl.ds(s * RPT, RPT)])
    plsc.subcore_barrier()

    def body(j, carry):
        pltpu.sync_copy(ones_v, acc.at[dst_v.at[j]], add=True)
        return carry

    lax.fori_loop(0, CPW, body, 0)
    plsc.subcore_barrier()
    pltpu.sync_copy(acc.at[pl.ds(s * RPT, RPT)], out.at[c, pl.ds(s * RPT, RPT)])


@functools.partial(
    pl.kernel,
    out_type=jax.ShapeDtypeStruct((NC, NP, D), jnp.float32),
    mesh=_mesh,
    scratch_types=[
        pltpu.VMEM((CPW // 2, K), jnp.int32),
        pltpu.VMEM((CPW // 2, K), jnp.int32),
        pltpu.VMEM((K, D), jnp.float32),
        pltpu.VMEM((K, D), jnp.float32),
        pltpu.VMEM_SHARED((NP, D), jnp.float32),
        pltpu.SemaphoreType.DMA,
        pltpu.SemaphoreType.DMA,
    ],
)
def _sc_propagate(u, src2d, dst2d, zerosd, out, src_v, dst_v,
                  rows_a, rows_b, acc, sem_a, sem_b):
    c = lax.axis_index("c")
    s = lax.axis_index("s")
    w = c * NS + s
    pltpu.sync_copy(zerosd, acc.at[pl.ds(s * RPT, RPT)])
    plsc.subcore_barrier()

    CPS = CPW // 2
    for st in range(2):
        pltpu.sync_copy(src2d.at[pl.ds(w * CPW + st * CPS, CPS)], src_v)
        pltpu.sync_copy(dst2d.at[pl.ds(w * CPW + st * CPS, CPS)], dst_v)
        pltpu.async_copy(u.at[src_v.at[0]], rows_a, sem_a)

        def body(t, carry):
            j0 = 2 * t
            j1 = j0 + 1
            j2 = jnp.minimum(j0 + 2, CPS - 1)
            pltpu.async_copy(u.at[src_v.at[j1]], rows_b, sem_b)
            pltpu.make_async_copy(u.at[src_v.at[j0]], rows_a, sem_a).wait()
            pltpu.sync_copy(rows_a, acc.at[dst_v.at[j0]], add=True)
            pltpu.async_copy(u.at[src_v.at[j2]], rows_a, sem_a)
            pltpu.make_async_copy(u.at[src_v.at[j1]], rows_b, sem_b).wait()
            pltpu.sync_copy(rows_b, acc.at[dst_v.at[j1]], add=True)
            return carry

        lax.fori_loop(0, CPS // 2, body, 0)
        pltpu.make_async_copy(u.at[src_v.at[CPS - 1]], rows_a, sem_a).wait()
    plsc.subcore_barrier()
    pltpu.sync_copy(acc.at[pl.ds(s * RPT, RPT)], out.at[c, pl.ds(s * RPT, RPT)])


def _tc_first(x_pad, W0, d0, d1):
    def body(x_ref, w_ref, d0_ref, d1_ref, u_ref, dis_ref):
        deg = d0_ref[:, :1] + d1_ref[:, :1]
        dis = jnp.where(deg > 0.0, lax.rsqrt(jnp.maximum(deg, 1.0)), 0.0)
        u_ref[...] = dis * jnp.dot(x_ref[...], w_ref[...],
                                   preferred_element_type=jnp.float32)
        dis_ref[...] = dis

    return pl.pallas_call(
        body,
        out_shape=[
            jax.ShapeDtypeStruct((NP, D), jnp.float32),
            jax.ShapeDtypeStruct((NP, 1), jnp.float32),
        ],
    )(x_pad, W0, d0, d1)


def _tc_mid(s0, s1, dis, b, W):
    def body(s0_ref, s1_ref, dis_ref, b_ref, w_ref, u_ref):
        dis = dis_ref[...]
        h = jnp.maximum(dis * (s0_ref[...] + s1_ref[...]) + b_ref[...], 0.0)
        u_ref[...] = dis * jnp.dot(h, w_ref[...],
                                   preferred_element_type=jnp.float32)

    return pl.pallas_call(
        body,
        out_shape=jax.ShapeDtypeStruct((NP, D), jnp.float32),
    )(s0, s1, dis, b, W)


def _tc_final(s0, s1, dis, b):
    def body(s0_ref, s1_ref, dis_ref, b_ref, o_ref):
        o_ref[...] = dis_ref[...] * (s0_ref[...] + s1_ref[...]) + b_ref[...]

    return pl.pallas_call(
        body,
        out_shape=jax.ShapeDtypeStruct((NP, D), jnp.float32),
    )(s0, s1, dis, b)


def kernel(x, edge_index, W0, b0, W1, b1, W2, b2, W3, b3):
    src = edge_index[0]
    dst = edge_index[1]
    pad = E_PAD - N_EDGES
    pad_iota = jnp.arange(pad, dtype=jnp.int32) % (NP - N_NODES)
    src2d = jnp.concatenate([src, N_NODES + pad_iota]).reshape(CT_PAD, K)
    dst2d = jnp.concatenate([dst, N_NODES + pad_iota]).reshape(CT_PAD, K)
    perm = jnp.arange(CT_PAD).reshape(CPW, NW).T.reshape(-1)
    src2d = src2d[perm]
    dst2d = dst2d[perm]

    onesd = jnp.tile(
        (jnp.arange(D) == 0).astype(jnp.float32)[None, :], (K, 1))
    zerosd = jnp.zeros((RPT, D), jnp.float32)
    x_pad = jnp.pad(x, ((0, NP - N_NODES), (0, 0)))

    degp = _sc_degree(dst2d, onesd, zerosd)
    u, dis = _tc_first(x_pad, W0, degp[0, :, :1], degp[1, :, :1])

    bs = [b0.reshape(1, D), b1.reshape(1, D), b2.reshape(1, D), b3.reshape(1, D)]
    Ws = [W1, W2, W3]
    for i in range(3):
        sp = _sc_propagate(u, src2d, dst2d, zerosd)
        u = _tc_mid(sp[0], sp[1], dis, bs[i], Ws[i])
    sp = _sc_propagate(u, src2d, dst2d, zerosd)
    out = _tc_final(sp[0], sp[1], dis, bs[3])
    return out[:N_NODES]
